# Initial kernel scaffold; baseline (speedup 1.0000x reference)
import functools

import jax
import jax.numpy as jnp
from jax import lax
from jax.experimental import pallas as pl
from jax.experimental.pallas import tpu as pltpu
from jax.experimental.pallas import tpu_sc as plsc

NC = 2
NS = 16
CHUNK = 128


def _pick_div(n, cap):
    for d in range(cap, 0, -1):
        if n % d == 0:
            return d
    return 1


def _zero_fill(ref, nrows):
    zv = jnp.zeros((16,), jnp.float32)

    def st(i, _):
        ref[i] = zv
        return 0

    lax.fori_loop(0, nrows, st, 0, unroll=8)


def _zero_fill_1d(ref, n):
    zv = jnp.zeros((16,), jnp.float32)

    def st(i, _):
        ref[pl.ds(i * 16, 16)] = zv
        return 0

    lax.fori_loop(0, n // 16, st, 0, unroll=8)


def _deg_body(np_, snp, nblk, ch, colm, ewm, out, acc, colb, ewb, zb):
    c = lax.axis_index("c")
    s = lax.axis_index("s")
    w = c * NS + s
    _zero_fill_1d(zb, snp)
    pltpu.sync_copy(zb, acc.at[pl.ds(s * snp, snp)])
    plsc.subcore_barrier()

    def blk(b, _):
        bb = (w * nblk + b) * ch
        pltpu.sync_copy(colm.at[pl.ds(bb, ch)], colb)
        pltpu.sync_copy(ewm.at[pl.ds(bb, ch)], ewb)
        for j in range(ch):
            pltpu.sync_copy(ewb.at[j], acc.at[colb.at[j]], add=True)
        return 0

    lax.fori_loop(0, nblk, blk, 0)
    plsc.subcore_barrier()
    pltpu.sync_copy(acc.at[pl.ds(s * snp, snp)], zb)
    pltpu.sync_copy(zb, out.at[c, pl.ds(s * snp, snp)])


def _make_deg(npad, epc):
    snp = npad // NS
    c_per_tile = epc // (NC * NS)
    ch = _pick_div(c_per_tile, 32)
    nblk = c_per_tile // ch
    mesh = plsc.VectorSubcoreMesh(core_axis_name="c", subcore_axis_name="s")
    return pl.kernel(
        functools.partial(_deg_body, npad, snp, nblk, ch),
        out_type=jax.ShapeDtypeStruct((NC, npad), jnp.float32),
        mesh=mesh,
        scratch_types=[
            pltpu.VMEM_SHARED((npad,), jnp.float32),
            pltpu.VMEM((ch, CHUNK), jnp.int32),
            pltpu.VMEM((ch, CHUNK), jnp.float32),
            pltpu.VMEM((snp,), jnp.float32),
        ],
    )


def _conv_body(n, rt, wb, nblk, ch, split_edges, rowm, colm, ewm, ytab, out,
               acc, rowb, colb, ewb, gath, zb, sem):
    c = lax.axis_index("c")
    s = lax.axis_index("s")
    _zero_fill(zb, wb)

    def zcp(i, _):
        pltpu.sync_copy(zb, acc.at[pl.ds(s * rt + i * wb, wb)])
        return 0

    lax.fori_loop(0, rt // wb, zcp, 0)
    plsc.subcore_barrier()

    def blk(b, _):
        if split_edges:
            bb = ((c * NS + s) * nblk + b) * ch
            pltpu.sync_copy(rowm.at[pl.ds(bb, ch)], rowb)
        else:
            bb = (s * nblk + b) * ch
            pltpu.sync_copy(rowm.at[c, pl.ds(bb, ch)], rowb)
        pltpu.sync_copy(colm.at[pl.ds(bb, ch)], colb)
        pltpu.sync_copy(ewm.at[pl.ds(bb, ch)], ewb)
        for j in range(ch):
            pltpu.async_copy(ytab.at[rowb.at[j]], gath, sem).wait()

            def scale(e, _):
                gath[e] = gath[e] * ewb[j, e]
                return 0

            lax.fori_loop(0, CHUNK, scale, 0, unroll=8)
            pltpu.sync_copy(gath, acc.at[colb.at[j]], add=True)
        return 0

    lax.fori_loop(0, nblk, blk, 0)
    plsc.subcore_barrier()

    def wbk(i, _):
        pltpu.sync_copy(acc.at[pl.ds(s * rt + i * wb, wb)], zb)
        pltpu.sync_copy(zb, out.at[pl.ds(c * n + s * rt + i * wb, wb)])
        return 0

    lax.fori_loop(0, rt // wb, wbk, 0)


def _make_conv(n, epc, split_edges):
    rt = n // NS
    wb = _pick_div(rt, 1024)
    c_per_tile = epc // (NC * NS) if split_edges else epc // NS
    ch = _pick_div(c_per_tile, 32)
    nblk = c_per_tile // ch
    mesh = plsc.VectorSubcoreMesh(core_axis_name="c", subcore_axis_name="s")
    row_shape = (epc, CHUNK) if split_edges else (NC, epc, CHUNK)
    del row_shape
    return pl.kernel(
        functools.partial(_conv_body, n, rt, wb, nblk, ch, split_edges),
        out_type=jax.ShapeDtypeStruct((NC * n, 16), jnp.float32),
        mesh=mesh,
        scratch_types=[
            pltpu.VMEM_SHARED((n, 16), jnp.float32),
            pltpu.VMEM((ch, CHUNK), jnp.int32),
            pltpu.VMEM((ch, CHUNK), jnp.int32),
            pltpu.VMEM((ch, CHUNK), jnp.float32),
            pltpu.VMEM((CHUNK, 16), jnp.float32),
            pltpu.VMEM((wb, 16), jnp.float32),
            pltpu.SemaphoreType.DMA,
        ],
    )


def _tc1_body(dega, degb, x, dinv_o, y1_o):
    deg = dega[...] + degb[...] + 1.0
    dinv = lax.rsqrt(deg)
    dinv_o[...] = dinv
    y1_o[...] = x[...] * dinv


def _ln(h, g, b):
    m = jnp.mean(h, axis=-1, keepdims=True)
    v = jnp.mean((h - m) ** 2, axis=-1, keepdims=True)
    return (h - m) * lax.rsqrt(v + 1e-5) * g + b


def _tc2_body(a1a, a1b, x, dinv, W1, b1, Wr1, br1, g1, be1, W2, Wr2, br2, b2,
              y2_o, res2_o):
    dv = dinv[...]
    conv_in = dv * (a1a[...] + a1b[...]) + (dv * dv) * x[...]
    h = (jnp.dot(conv_in, W1[...], preferred_element_type=jnp.float32)
         + b1[...]
         + jnp.dot(x[...], Wr1[...], preferred_element_type=jnp.float32)
         + br1[...])
    x1 = jnp.maximum(_ln(h, g1[...], be1[...]), 0.0)
    y2 = dv * jnp.dot(x1, W2[...], preferred_element_type=jnp.float32)
    y2_o[0] = y2[:, :16]
    y2_o[1] = y2[:, 16:]
    res2_o[...] = (jnp.dot(x1, Wr2[...], preferred_element_type=jnp.float32)
                   + br2[...] + b2[...])


def _tc3_body(agg2, y2, res2, dinv, g2, be2, Wfc, bfc, Wp, bp, log_o, emb_o):
    dv = dinv[...]
    a = jnp.concatenate([agg2[0], agg2[1]], axis=1)
    yy = jnp.concatenate([y2[0], y2[1]], axis=1)
    h = dv * a + dv * yy + res2[...]
    x2 = jnp.maximum(_ln(h, g2[...], be2[...]), 0.0)
    log_o[...] = (jnp.dot(x2, Wfc[...], preferred_element_type=jnp.float32)
                  + bfc[...])
    p = jnp.dot(x2, Wp[...], preferred_element_type=jnp.float32) + bp[...]
    nrm = jnp.sqrt(jnp.sum(p * p, axis=-1, keepdims=True))
    emb_o[...] = p / jnp.maximum(nrm, 1e-12)


def _row_spec(bn, f):
    return pl.BlockSpec((bn, f), lambda i: (i, 0))


def _full_spec(shape):
    nd = len(shape)
    return pl.BlockSpec(shape, lambda i: (0,) * nd)


def kernel(x, edge_index, edge_weight, W1, b1, Wr1, br1, g1, be1,
           W2, b2, Wr2, br2, g2, be2, Wfc, bfc, Wp, bp):
    n, in_dim = x.shape
    e = edge_weight.shape[0]
    hid = W1.shape[1]
    r = W2.shape[1]
    emb_dim = Wp.shape[1]

    grp = CHUNK * NC * NS
    ep = ((e + grp - 1) // grp) * grp
    padn = ep - e
    row = edge_index[0]
    col = edge_index[1]
    if padn:
        pad_idx = (jnp.arange(padn, dtype=jnp.int32) * 97) % n
        row = jnp.concatenate([row, pad_idx])
        col = jnp.concatenate([col, pad_idx])
        ew = jnp.concatenate([edge_weight, jnp.zeros((padn,), jnp.float32)])
    else:
        ew = edge_weight
    epc = ep // CHUNK
    rowm = row.reshape(epc, CHUNK)
    colm = col.reshape(epc, CHUNK)
    ewm = ew.reshape(epc, CHUNK)
    rowm2 = jnp.stack([rowm, rowm + n])

    npad = ((n + NS * 8 - 1) // (NS * 8)) * NS * 8

    deg2 = _make_deg(npad, epc)(colm, ewm)
    dega = deg2[0, :n].reshape(n, 1)
    degb = deg2[1, :n].reshape(n, 1)

    bn = _pick_div(n, 2500)
    grid = n // bn
    dinv, y1 = pl.pallas_call(
        _tc1_body,
        grid=(grid,),
        in_specs=[_row_spec(bn, 1), _row_spec(bn, 1), _row_spec(bn, in_dim)],
        out_specs=[_row_spec(bn, 1), _row_spec(bn, in_dim)],
        out_shape=[jax.ShapeDtypeStruct((n, 1), jnp.float32),
                   jax.ShapeDtypeStruct((n, in_dim), jnp.float32)],
    )(dega, degb, x)

    agg1 = _make_conv(n, epc, split_edges=True)(rowm, colm, ewm, y1)

    y2, res2 = pl.pallas_call(
        _tc2_body,
        grid=(grid,),
        in_specs=[_row_spec(bn, in_dim), _row_spec(bn, in_dim),
                  _row_spec(bn, in_dim), _row_spec(bn, 1),
                  _full_spec((in_dim, hid)), _full_spec((1, hid)),
                  _full_spec((in_dim, hid)), _full_spec((1, hid)),
                  _full_spec((1, hid)), _full_spec((1, hid)),
                  _full_spec((hid, r)), _full_spec((hid, r)),
                  _full_spec((1, r)), _full_spec((1, r))],
        out_specs=[pl.BlockSpec((2, bn, 16), lambda i: (0, i, 0)),
                   _row_spec(bn, r)],
        out_shape=[jax.ShapeDtypeStruct((2, n, 16), jnp.float32),
                   jax.ShapeDtypeStruct((n, r), jnp.float32)],
    )(agg1[:n], agg1[n:], x, dinv,
      W1, b1.reshape(1, -1), Wr1, br1.reshape(1, -1),
      g1.reshape(1, -1), be1.reshape(1, -1),
      W2, Wr2, br2.reshape(1, -1), b2.reshape(1, -1))

    agg2 = _make_conv(n, epc, split_edges=False)(
        rowm2, colm, ewm, y2.reshape(2 * n, 16))

    logits, emb = pl.pallas_call(
        _tc3_body,
        grid=(grid,),
        in_specs=[pl.BlockSpec((2, bn, 16), lambda i: (0, i, 0)),
                  pl.BlockSpec((2, bn, 16), lambda i: (0, i, 0)),
                  _row_spec(bn, r), _row_spec(bn, 1),
                  _full_spec((1, r)), _full_spec((1, r)),
                  _full_spec((r, 1)), _full_spec((1, 1)),
                  _full_spec((r, emb_dim)), _full_spec((1, emb_dim))],
        out_specs=[_row_spec(bn, 1), _row_spec(bn, emb_dim)],
        out_shape=[jax.ShapeDtypeStruct((n, 1), jnp.float32),
                   jax.ShapeDtypeStruct((n, emb_dim), jnp.float32)],
    )(agg2.reshape(2, n, 16), y2, res2, dinv,
      g2.reshape(1, -1), be2.reshape(1, -1),
      Wfc, bfc.reshape(1, -1), Wp, bp.reshape(1, -1))

    return logits.reshape(n), emb


# speedup vs baseline: 12.0438x; 12.0438x over previous
import functools

import jax
import jax.numpy as jnp
from jax import lax
from jax.experimental import pallas as pl
from jax.experimental.pallas import tpu as pltpu
from jax.experimental.pallas import tpu_sc as plsc

NC = 2
NS = 16
CHUNK = 128


def _pick_div8(n, cap):
    for d in range(cap - cap % 8, 0, -8):
        if n % d == 0:
            return d
    raise ValueError(f"no multiple-of-8 divisor of {n}")


def _zero_fill(ref, nrows):
    zv = jnp.zeros((16,), jnp.float32)

    def st(i, _):
        ref[i] = zv
        return 0

    lax.fori_loop(0, nrows, st, 0, unroll=8)


def _zero_fill_1d(ref, n):
    zv = jnp.zeros((16,), jnp.float32)

    def st(i, _):
        ref[pl.ds(i * 16, 16)] = zv
        return 0

    lax.fori_loop(0, n // 16, st, 0, unroll=8)


def _deg_body(snp, nblk, ch, colm, ewm, out, acc, colb, ewb, zb):
    c = lax.axis_index("c")
    s = lax.axis_index("s")
    w = c * NS + s
    _zero_fill_1d(zb, snp)
    pltpu.sync_copy(zb, acc.at[pl.ds(s * snp, snp)])
    plsc.subcore_barrier()

    def blk(b, _):
        bb = (w * nblk + b) * ch
        pltpu.sync_copy(colm.at[pl.ds(bb, ch)], colb)
        pltpu.sync_copy(ewm.at[pl.ds(bb, ch)], ewb)
        for j in range(ch):
            pltpu.sync_copy(ewb.at[j], acc.at[colb.at[j]], add=True)
        return 0

    lax.fori_loop(0, nblk, blk, 0)
    plsc.subcore_barrier()
    pltpu.sync_copy(acc.at[pl.ds(s * snp, snp)], zb)
    pltpu.sync_copy(zb, out.at[c, pl.ds(s * snp, snp)])


def _make_deg(npad, epc):
    snp = npad // NS
    c_per_tile = epc // (NC * NS)
    ch = _pick_div8(c_per_tile, 32)
    nblk = c_per_tile // ch
    mesh = plsc.VectorSubcoreMesh(core_axis_name="c", subcore_axis_name="s")
    return pl.kernel(
        functools.partial(_deg_body, snp, nblk, ch),
        out_type=jax.ShapeDtypeStruct((NC, npad), jnp.float32),
        mesh=mesh,
        compiler_params=pltpu.CompilerParams(use_tc_tiling_on_sc=False),
        scratch_types=[
            pltpu.VMEM_SHARED((npad,), jnp.float32),
            pltpu.VMEM((ch, CHUNK), jnp.int32),
            pltpu.VMEM((ch, CHUNK), jnp.float32),
            pltpu.VMEM((snp,), jnp.float32),
        ],
    )


def _conv_body(half, rt, nblk, ch, rowm, colm, ewm, ytab, out,
               acc, rowb, colb, ewb, gath, zb, sem):
    c = lax.axis_index("c")
    s = lax.axis_index("s")
    base = c * half
    trash = half + jnp.arange(16, dtype=jnp.int32)
    _zero_fill(zb, rt)
    pltpu.sync_copy(zb, acc.at[pl.ds(s * rt, rt)])
    plsc.subcore_barrier()

    def blk(b, _):
        bb = (s * nblk + b) * ch
        pltpu.sync_copy(rowm.at[pl.ds(bb, ch)], rowb)
        pltpu.sync_copy(colm.at[pl.ds(bb, ch)], colb)
        pltpu.sync_copy(ewm.at[pl.ds(bb, ch)], ewb)
        for j in range(ch):

            def xform(k, _):
                cv = colb[j, pl.ds(k * 16, 16)] - base
                ok = (cv >= 0) & (cv < half)
                colb[j, pl.ds(k * 16, 16)] = jnp.where(ok, cv, trash)
                return 0

            lax.fori_loop(0, CHUNK // 16, xform, 0)
            pltpu.async_copy(ytab.at[rowb.at[j]], gath, sem).wait()

            def scale(k, _):
                ewv = ewb[j, pl.ds(k * 16, 16)]
                kb = k * 16
                for i in range(16):
                    gath[kb + i] = gath[kb + i] * ewv[i]
                return 0

            lax.fori_loop(0, CHUNK // 16, scale, 0)
            pltpu.sync_copy(gath, acc.at[colb.at[j]], add=True)
        return 0

    lax.fori_loop(0, nblk, blk, 0)
    plsc.subcore_barrier()
    pltpu.sync_copy(acc.at[pl.ds(s * rt, rt)], zb)
    pltpu.sync_copy(zb, out.at[pl.ds(c * half + s * rt, rt)])


def _make_conv(n2, epc):
    half = n2 // NC
    rt = half // NS
    c_per_tile = epc // NS
    ch = _pick_div8(c_per_tile, 32)
    nblk = c_per_tile // ch
    mesh = plsc.VectorSubcoreMesh(core_axis_name="c", subcore_axis_name="s")
    return pl.kernel(
        functools.partial(_conv_body, half, rt, nblk, ch),
        out_type=jax.ShapeDtypeStruct((n2, 16), jnp.float32),
        mesh=mesh,
        compiler_params=pltpu.CompilerParams(use_tc_tiling_on_sc=False),
        scratch_types=[
            pltpu.VMEM_SHARED((half + 16, 16), jnp.float32),
            pltpu.VMEM((ch, CHUNK), jnp.int32),
            pltpu.VMEM((ch, CHUNK), jnp.int32),
            pltpu.VMEM((ch, CHUNK), jnp.float32),
            pltpu.VMEM((CHUNK, 16), jnp.float32),
            pltpu.VMEM((rt, 16), jnp.float32),
            pltpu.SemaphoreType.DMA,
        ],
    )


def _tc1_body(dega, degb, x, dinv_o, y1_o):
    deg = dega[...] + degb[...] + 1.0
    dinv = lax.rsqrt(deg)
    dinv_o[...] = dinv
    y1_o[...] = x[...] * dinv


def _ln(h, g, b):
    m = jnp.mean(h, axis=-1, keepdims=True)
    v = jnp.mean((h - m) ** 2, axis=-1, keepdims=True)
    return (h - m) * lax.rsqrt(v + 1e-5) * g + b


def _tc2_body(a1, x, dinv, W1, b1, Wr1, br1, g1, be1, W2, Wr2, br2, b2,
              y2_o, res2_o):
    dv = dinv[...]
    conv_in = dv * a1[...] + (dv * dv) * x[...]
    h = (jnp.dot(conv_in, W1[...], preferred_element_type=jnp.float32)
         + b1[...]
         + jnp.dot(x[...], Wr1[...], preferred_element_type=jnp.float32)
         + br1[...])
    x1 = jnp.maximum(_ln(h, g1[...], be1[...]), 0.0)
    y2 = dv * jnp.dot(x1, W2[...], preferred_element_type=jnp.float32)
    y2_o[0] = y2[:, :16]
    y2_o[1] = y2[:, 16:]
    res2_o[...] = (jnp.dot(x1, Wr2[...], preferred_element_type=jnp.float32)
                   + br2[...] + b2[...])


def _tc3_body(lo, hi, y2, res2, dinv, g2, be2, Wfc, bfc, Wp, bp,
              log_o, emb_o):
    dv = dinv[...]
    a = jnp.concatenate([lo[...], hi[...]], axis=1)
    yy = jnp.concatenate([y2[0], y2[1]], axis=1)
    h = dv * a + dv * yy + res2[...]
    x2 = jnp.maximum(_ln(h, g2[...], be2[...]), 0.0)
    log_o[...] = (jnp.dot(x2, Wfc[...], preferred_element_type=jnp.float32)
                  + bfc[...])
    p = jnp.dot(x2, Wp[...], preferred_element_type=jnp.float32) + bp[...]
    nrm = jnp.sqrt(jnp.sum(p * p, axis=-1, keepdims=True))
    emb_o[...] = p / jnp.maximum(nrm, 1e-12)


def _row_spec(bn, f):
    return pl.BlockSpec((bn, f), lambda i: (i, 0))


def _full_spec(shape):
    nd = len(shape)
    return pl.BlockSpec(shape, lambda i: (0,) * nd)


def kernel(x, edge_index, edge_weight, W1, b1, Wr1, br1, g1, be1,
           W2, b2, Wr2, br2, g2, be2, Wfc, bfc, Wp, bp):
    n, in_dim = x.shape
    e = edge_weight.shape[0]
    hid = W1.shape[1]
    r = W2.shape[1]
    emb_dim = Wp.shape[1]

    grp = CHUNK * NC * NS * 8
    ep = ((e + grp - 1) // grp) * grp
    padn = ep - e
    row = edge_index[0]
    col = edge_index[1]
    if padn:
        pad_idx = (jnp.arange(padn, dtype=jnp.int32) * 97) % n
        row = jnp.concatenate([row, pad_idx])
        col = jnp.concatenate([col, pad_idx])
        ew = jnp.concatenate([edge_weight, jnp.zeros((padn,), jnp.float32)])
    else:
        ew = edge_weight
    epc = ep // CHUNK
    rowm = row.reshape(epc, CHUNK)
    colm = col.reshape(epc, CHUNK)
    ewm = ew.reshape(epc, CHUNK)

    npad = ((n + NS * 128 - 1) // (NS * 128)) * NS * 128
    n2 = ((n + 2 * NS * 8 - 1) // (2 * NS * 8)) * 2 * NS * 8

    deg2 = _make_deg(npad, epc)(colm, ewm)
    dega = deg2[0, :n].reshape(n, 1)
    degb = deg2[1, :n].reshape(n, 1)

    bn = _pick_div8(n, 2048)
    grid = n // bn
    dinv, y1 = pl.pallas_call(
        _tc1_body,
        grid=(grid,),
        in_specs=[_row_spec(bn, 1), _row_spec(bn, 1), _row_spec(bn, in_dim)],
        out_specs=[_row_spec(bn, 1), _row_spec(bn, in_dim)],
        out_shape=[jax.ShapeDtypeStruct((n, 1), jnp.float32),
                   jax.ShapeDtypeStruct((n, in_dim), jnp.float32)],
    )(dega, degb, x)

    conv = _make_conv(n2, epc)
    agg1 = conv(rowm, colm, ewm, y1)

    y2, res2 = pl.pallas_call(
        _tc2_body,
        grid=(grid,),
        in_specs=[_row_spec(bn, in_dim),
                  _row_spec(bn, in_dim), _row_spec(bn, 1),
                  _full_spec((in_dim, hid)), _full_spec((1, hid)),
                  _full_spec((in_dim, hid)), _full_spec((1, hid)),
                  _full_spec((1, hid)), _full_spec((1, hid)),
                  _full_spec((hid, r)), _full_spec((hid, r)),
                  _full_spec((1, r)), _full_spec((1, r))],
        out_specs=[pl.BlockSpec((2, bn, 16), lambda i: (0, i, 0)),
                   _row_spec(bn, r)],
        out_shape=[jax.ShapeDtypeStruct((2, n, 16), jnp.float32),
                   jax.ShapeDtypeStruct((n, r), jnp.float32)],
    )(agg1[:n], x, dinv,
      W1, b1.reshape(1, -1), Wr1, br1.reshape(1, -1),
      g1.reshape(1, -1), be1.reshape(1, -1),
      W2, Wr2, br2.reshape(1, -1), b2.reshape(1, -1))

    agg2lo = conv(rowm, colm, ewm, y2[0])
    agg2hi = conv(rowm, colm, ewm, y2[1])

    logits, emb = pl.pallas_call(
        _tc3_body,
        grid=(grid,),
        in_specs=[_row_spec(bn, 16), _row_spec(bn, 16),
                  pl.BlockSpec((2, bn, 16), lambda i: (0, i, 0)),
                  _row_spec(bn, r), _row_spec(bn, 1),
                  _full_spec((1, r)), _full_spec((1, r)),
                  _full_spec((r, 1)), _full_spec((1, 1)),
                  _full_spec((r, emb_dim)), _full_spec((1, emb_dim))],
        out_specs=[_row_spec(bn, 1), _row_spec(bn, emb_dim)],
        out_shape=[jax.ShapeDtypeStruct((n, 1), jnp.float32),
                   jax.ShapeDtypeStruct((n, emb_dim), jnp.float32)],
    )(agg2lo[:n], agg2hi[:n], y2, res2, dinv,
      g2.reshape(1, -1), be2.reshape(1, -1),
      Wfc, bfc.reshape(1, -1), Wp, bp.reshape(1, -1))

    return logits.reshape(n), emb


# speedup vs baseline: 18.4107x; 1.5287x over previous
import functools

import jax
import jax.numpy as jnp
from jax import lax
from jax.experimental import pallas as pl
from jax.experimental.pallas import tpu as pltpu
from jax.experimental.pallas import tpu_sc as plsc

NC = 2
NS = 16
CHUNK = 128


def _pick_div8(n, cap):
    for d in range(cap - cap % 8, 0, -8):
        if n % d == 0:
            return d
    raise ValueError(f"no multiple-of-8 divisor of {n}")


def _zero_fill(ref, nrows):
    zv = jnp.zeros((16,), jnp.float32)

    def st(i, _):
        ref[i] = zv
        return 0

    lax.fori_loop(0, nrows, st, 0, unroll=8)


def _zero_fill_1d(ref, n):
    zv = jnp.zeros((16,), jnp.float32)

    def st(i, _):
        ref[pl.ds(i * 16, 16)] = zv
        return 0

    lax.fori_loop(0, n // 16, st, 0, unroll=8)


def _deg_body(snp, nblk, ch, colm, ewm, out, acc, colb, ewb, zb):
    c = lax.axis_index("c")
    s = lax.axis_index("s")
    w = c * NS + s
    _zero_fill_1d(zb, snp)
    pltpu.sync_copy(zb, acc.at[pl.ds(s * snp, snp)])
    plsc.subcore_barrier()

    def blk(b, _):
        bb = (w * nblk + b) * ch
        pltpu.sync_copy(colm.at[pl.ds(bb, ch)], colb)
        pltpu.sync_copy(ewm.at[pl.ds(bb, ch)], ewb)
        for j in range(ch):
            pltpu.sync_copy(ewb.at[j], acc.at[colb.at[j]], add=True)
        return 0

    lax.fori_loop(0, nblk, blk, 0)
    plsc.subcore_barrier()
    pltpu.sync_copy(acc.at[pl.ds(s * snp, snp)], zb)
    pltpu.sync_copy(zb, out.at[c, pl.ds(s * snp, snp)])


def _make_deg(npad, epc):
    snp = npad // NS
    c_per_tile = epc // (NC * NS)
    ch = _pick_div8(c_per_tile, 32)
    nblk = c_per_tile // ch
    mesh = plsc.VectorSubcoreMesh(core_axis_name="c", subcore_axis_name="s")
    return pl.kernel(
        functools.partial(_deg_body, snp, nblk, ch),
        out_type=jax.ShapeDtypeStruct((NC, npad), jnp.float32),
        mesh=mesh,
        compiler_params=pltpu.CompilerParams(use_tc_tiling_on_sc=False),
        scratch_types=[
            pltpu.VMEM_SHARED((npad,), jnp.float32),
            pltpu.VMEM((ch, CHUNK), jnp.int32),
            pltpu.VMEM((ch, CHUNK), jnp.float32),
            pltpu.VMEM((snp,), jnp.float32),
        ],
    )


def _conv_body(half, rt, nblk, ch, rowm, colm, ewm, ytab, out,
               acc, rowb, colb, ewb, gath_a, gath_b, zb,
               sem_i, sem_ga, sem_gb, sem_sa, sem_sb):
    c = lax.axis_index("c")
    s = lax.axis_index("s")
    base = c * half
    _zero_fill(zb, rt)
    pltpu.sync_copy(zb, acc.at[pl.ds(s * rt, rt)])
    plsc.subcore_barrier()
    gath = (gath_a, gath_b)
    sg = (sem_ga, sem_gb)
    ss = (sem_sa, sem_sb)

    def blk(b, _):
        bb = (s * nblk + b) * ch
        d1 = pltpu.async_copy(rowm.at[pl.ds(bb, ch)], rowb, sem_i)
        d2 = pltpu.async_copy(colm.at[pl.ds(bb, ch)], colb, sem_i)
        d3 = pltpu.async_copy(ewm.at[pl.ds(bb, ch)], ewb, sem_i)
        d1.wait()
        d2.wait()
        d3.wait()
        for j in range(ch):

            def xform(k, _):
                sl = pl.ds(k * 16, 16)
                cv = colb[j, sl] - base
                ok = (cv >= 0) & (cv < half)
                cv = jnp.where(cv < 0, cv + half, cv)
                cv = jnp.where(cv >= half, cv - half, cv)
                colb[j, sl] = cv
                ewb[j, sl] = jnp.where(ok, ewb[j, sl], 0.0)
                return 0

            lax.fori_loop(0, CHUNK // 16, xform, 0)
        dg = [pltpu.async_copy(ytab.at[rowb.at[0]], gath_a, sem_ga), None]
        ds_ = [None, None]
        for j in range(ch):
            p = j % 2
            q = 1 - p
            if ds_[q] is not None:
                ds_[q].wait()
                ds_[q] = None
            if j + 1 < ch:
                dg[q] = pltpu.async_copy(ytab.at[rowb.at[j + 1]], gath[q],
                                         sg[q])
            dg[p].wait()

            def scale(k, _):
                ewv = ewb[j, pl.ds(k * 16, 16)]
                kb = k * 16
                for i in range(16):
                    gath[p][kb + i] = gath[p][kb + i] * ewv[i]
                return 0

            lax.fori_loop(0, CHUNK // 16, scale, 0)
            ds_[p] = pltpu.async_copy(gath[p], acc.at[colb.at[j]], ss[p],
                                      add=True)
        for d in ds_:
            if d is not None:
                d.wait()
        return 0

    lax.fori_loop(0, nblk, blk, 0)
    plsc.subcore_barrier()
    pltpu.sync_copy(acc.at[pl.ds(s * rt, rt)], zb)
    pltpu.sync_copy(zb, out.at[pl.ds(c * half + s * rt, rt)])


def _make_conv(n2, epc):
    half = n2 // NC
    rt = half // NS
    c_per_tile = epc // NS
    ch = _pick_div8(c_per_tile, 32)
    nblk = c_per_tile // ch
    mesh = plsc.VectorSubcoreMesh(core_axis_name="c", subcore_axis_name="s")
    return pl.kernel(
        functools.partial(_conv_body, half, rt, nblk, ch),
        out_type=jax.ShapeDtypeStruct((n2, 16), jnp.float32),
        mesh=mesh,
        compiler_params=pltpu.CompilerParams(use_tc_tiling_on_sc=False),
        scratch_types=[
            pltpu.VMEM_SHARED((half, 16), jnp.float32),
            pltpu.VMEM((ch, CHUNK), jnp.int32),
            pltpu.VMEM((ch, CHUNK), jnp.int32),
            pltpu.VMEM((ch, CHUNK), jnp.float32),
            pltpu.VMEM((CHUNK, 16), jnp.float32),
            pltpu.VMEM((CHUNK, 16), jnp.float32),
            pltpu.VMEM((rt, 16), jnp.float32),
            pltpu.SemaphoreType.DMA,
            pltpu.SemaphoreType.DMA,
            pltpu.SemaphoreType.DMA,
            pltpu.SemaphoreType.DMA,
            pltpu.SemaphoreType.DMA,
        ],
    )


def _tc1_body(dega, degb, x, dinv_o, y1_o):
    deg = dega[...] + degb[...] + 1.0
    dinv = lax.rsqrt(deg)
    dinv_o[...] = dinv
    y1_o[...] = x[...] * dinv


def _ln(h, g, b):
    m = jnp.mean(h, axis=-1, keepdims=True)
    v = jnp.mean((h - m) ** 2, axis=-1, keepdims=True)
    return (h - m) * lax.rsqrt(v + 1e-5) * g + b


def _tc2_body(a1, x, dinv, W1, b1, Wr1, br1, g1, be1, W2, Wr2, br2, b2,
              y2_o, res2_o):
    dv = dinv[...]
    conv_in = dv * a1[...] + (dv * dv) * x[...]
    h = (jnp.dot(conv_in, W1[...], preferred_element_type=jnp.float32)
         + b1[...]
         + jnp.dot(x[...], Wr1[...], preferred_element_type=jnp.float32)
         + br1[...])
    x1 = jnp.maximum(_ln(h, g1[...], be1[...]), 0.0)
    y2 = dv * jnp.dot(x1, W2[...], preferred_element_type=jnp.float32)
    y2_o[0] = y2[:, :16]
    y2_o[1] = y2[:, 16:]
    res2_o[...] = (jnp.dot(x1, Wr2[...], preferred_element_type=jnp.float32)
                   + br2[...] + b2[...])


def _tc3_body(lo, hi, y2, res2, dinv, g2, be2, Wfc, bfc, Wp, bp,
              log_o, emb_o):
    dv = dinv[...]
    a = jnp.concatenate([lo[...], hi[...]], axis=1)
    yy = jnp.concatenate([y2[0], y2[1]], axis=1)
    h = dv * a + dv * yy + res2[...]
    x2 = jnp.maximum(_ln(h, g2[...], be2[...]), 0.0)
    log_o[...] = (jnp.dot(x2, Wfc[...], preferred_element_type=jnp.float32)
                  + bfc[...])
    p = jnp.dot(x2, Wp[...], preferred_element_type=jnp.float32) + bp[...]
    nrm = jnp.sqrt(jnp.sum(p * p, axis=-1, keepdims=True))
    emb_o[...] = p / jnp.maximum(nrm, 1e-12)


def _row_spec(bn, f):
    return pl.BlockSpec((bn, f), lambda i: (i, 0))


def _full_spec(shape):
    nd = len(shape)
    return pl.BlockSpec(shape, lambda i: (0,) * nd)


def kernel(x, edge_index, edge_weight, W1, b1, Wr1, br1, g1, be1,
           W2, b2, Wr2, br2, g2, be2, Wfc, bfc, Wp, bp):
    n, in_dim = x.shape
    e = edge_weight.shape[0]
    hid = W1.shape[1]
    r = W2.shape[1]
    emb_dim = Wp.shape[1]

    grp = CHUNK * NC * NS * 8
    ep = ((e + grp - 1) // grp) * grp
    padn = ep - e
    row = edge_index[0]
    col = edge_index[1]
    if padn:
        pad_idx = (jnp.arange(padn, dtype=jnp.int32) * 97) % n
        row = jnp.concatenate([row, pad_idx])
        col = jnp.concatenate([col, pad_idx])
        ew = jnp.concatenate([edge_weight, jnp.zeros((padn,), jnp.float32)])
    else:
        ew = edge_weight
    epc = ep // CHUNK
    rowm = row.reshape(epc, CHUNK)
    colm = col.reshape(epc, CHUNK)
    ewm = ew.reshape(epc, CHUNK)

    npad = ((n + NS * 128 - 1) // (NS * 128)) * NS * 128
    n2 = ((n + 2 * NS * 8 - 1) // (2 * NS * 8)) * 2 * NS * 8

    deg2 = _make_deg(npad, epc)(colm, ewm)
    dega = deg2[0, :n].reshape(n, 1)
    degb = deg2[1, :n].reshape(n, 1)

    bn = _pick_div8(n, 2048)
    grid = n // bn
    dinv, y1 = pl.pallas_call(
        _tc1_body,
        grid=(grid,),
        in_specs=[_row_spec(bn, 1), _row_spec(bn, 1), _row_spec(bn, in_dim)],
        out_specs=[_row_spec(bn, 1), _row_spec(bn, in_dim)],
        out_shape=[jax.ShapeDtypeStruct((n, 1), jnp.float32),
                   jax.ShapeDtypeStruct((n, in_dim), jnp.float32)],
    )(dega, degb, x)

    conv = _make_conv(n2, epc)
    agg1 = conv(rowm, colm, ewm, y1)

    y2, res2 = pl.pallas_call(
        _tc2_body,
        grid=(grid,),
        in_specs=[_row_spec(bn, in_dim),
                  _row_spec(bn, in_dim), _row_spec(bn, 1),
                  _full_spec((in_dim, hid)), _full_spec((1, hid)),
                  _full_spec((in_dim, hid)), _full_spec((1, hid)),
                  _full_spec((1, hid)), _full_spec((1, hid)),
                  _full_spec((hid, r)), _full_spec((hid, r)),
                  _full_spec((1, r)), _full_spec((1, r))],
        out_specs=[pl.BlockSpec((2, bn, 16), lambda i: (0, i, 0)),
                   _row_spec(bn, r)],
        out_shape=[jax.ShapeDtypeStruct((2, n, 16), jnp.float32),
                   jax.ShapeDtypeStruct((n, r), jnp.float32)],
    )(agg1[:n], x, dinv,
      W1, b1.reshape(1, -1), Wr1, br1.reshape(1, -1),
      g1.reshape(1, -1), be1.reshape(1, -1),
      W2, Wr2, br2.reshape(1, -1), b2.reshape(1, -1))

    agg2lo = conv(rowm, colm, ewm, y2[0])
    agg2hi = conv(rowm, colm, ewm, y2[1])

    logits, emb = pl.pallas_call(
        _tc3_body,
        grid=(grid,),
        in_specs=[_row_spec(bn, 16), _row_spec(bn, 16),
                  pl.BlockSpec((2, bn, 16), lambda i: (0, i, 0)),
                  _row_spec(bn, r), _row_spec(bn, 1),
                  _full_spec((1, r)), _full_spec((1, r)),
                  _full_spec((r, 1)), _full_spec((1, 1)),
                  _full_spec((r, emb_dim)), _full_spec((1, emb_dim))],
        out_specs=[_row_spec(bn, 1), _row_spec(bn, emb_dim)],
        out_shape=[jax.ShapeDtypeStruct((n, 1), jnp.float32),
                   jax.ShapeDtypeStruct((n, emb_dim), jnp.float32)],
    )(agg2lo[:n], agg2hi[:n], y2, res2, dinv,
      g2.reshape(1, -1), be2.reshape(1, -1),
      Wfc, bfc.reshape(1, -1), Wp, bp.reshape(1, -1))

    return logits.reshape(n), emb


# speedup vs baseline: 25.0843x; 1.3625x over previous
import functools

import jax
import jax.numpy as jnp
from jax import lax
from jax.experimental import pallas as pl
from jax.experimental.pallas import tpu as pltpu
from jax.experimental.pallas import tpu_sc as plsc

NC = 2
NS = 16
CHUNK = 128


def _pick_div8(n, cap):
    for d in range(cap - cap % 8, 0, -8):
        if n % d == 0:
            return d
    raise ValueError(f"no multiple-of-8 divisor of {n}")


def _zero_fill(ref, nrows):
    zv = jnp.zeros((16,), jnp.float32)

    def st(i, _):
        ref[i] = zv
        return 0

    lax.fori_loop(0, nrows, st, 0, unroll=8)


def _zero_fill_1d(ref, n):
    zv = jnp.zeros((16,), jnp.float32)

    def st(i, _):
        ref[pl.ds(i * 16, 16)] = zv
        return 0

    lax.fori_loop(0, n // 16, st, 0, unroll=8)


def _deg_body(snp, nblk, ch, colm, ewm, out, acc, colb, ewb, zb):
    c = lax.axis_index("c")
    s = lax.axis_index("s")
    w = c * NS + s
    _zero_fill_1d(zb, snp)
    pltpu.sync_copy(zb, acc.at[pl.ds(s * snp, snp)])
    plsc.subcore_barrier()

    def blk(b, _):
        bb = (w * nblk + b) * ch
        pltpu.sync_copy(colm.at[pl.ds(bb, ch)], colb)
        pltpu.sync_copy(ewm.at[pl.ds(bb, ch)], ewb)
        for j in range(ch):
            pltpu.sync_copy(ewb.at[j], acc.at[colb.at[j]], add=True)
        return 0

    lax.fori_loop(0, nblk, blk, 0)
    plsc.subcore_barrier()
    pltpu.sync_copy(acc.at[pl.ds(s * snp, snp)], zb)
    pltpu.sync_copy(zb, out.at[c, pl.ds(s * snp, snp)])


def _make_deg(npad, epc):
    snp = npad // NS
    c_per_tile = epc // (NC * NS)
    ch = _pick_div8(c_per_tile, 32)
    nblk = c_per_tile // ch
    mesh = plsc.VectorSubcoreMesh(core_axis_name="c", subcore_axis_name="s")
    return pl.kernel(
        functools.partial(_deg_body, snp, nblk, ch),
        out_type=jax.ShapeDtypeStruct((NC, npad), jnp.float32),
        mesh=mesh,
        compiler_params=pltpu.CompilerParams(use_tc_tiling_on_sc=False),
        scratch_types=[
            pltpu.VMEM_SHARED((npad,), jnp.float32),
            pltpu.VMEM((ch, CHUNK), jnp.int32),
            pltpu.VMEM((ch, CHUNK), jnp.float32),
            pltpu.VMEM((snp,), jnp.float32),
        ],
    )


NBUF = 4
LOOKAHEAD = 2


def _conv_body(half, rt, nblk, ch, rowm, colm, ewm, ytab, out,
               acc, rowb, colb, ewb, g0, g1, g2, g3, zb, sem_i,
               sg0, sg1, sg2, sg3, ss0, ss1, ss2, ss3):
    c = lax.axis_index("c")
    s = lax.axis_index("s")
    base = c * half
    _zero_fill(zb, rt)
    pltpu.sync_copy(zb, acc.at[pl.ds(s * rt, rt)])
    plsc.subcore_barrier()
    gath = (g0, g1, g2, g3)
    sg = (sg0, sg1, sg2, sg3)
    ss = (ss0, ss1, ss2, ss3)

    def blk(b, _):
        bb = (s * nblk + b) * ch
        d1 = pltpu.async_copy(rowm.at[pl.ds(bb, ch)], rowb, sem_i)
        d2 = pltpu.async_copy(colm.at[pl.ds(bb, ch)], colb, sem_i)
        d3 = pltpu.async_copy(ewm.at[pl.ds(bb, ch)], ewb, sem_i)
        d1.wait()
        d2.wait()
        d3.wait()
        for j in range(ch):

            def xform(k, _):
                sl = pl.ds(k * 16, 16)
                cv = colb[j, sl] - base
                ok = (cv >= 0) & (cv < half)
                cv = jnp.where(cv < 0, cv + half, cv)
                cv = jnp.where(cv >= half, cv - half, cv)
                colb[j, sl] = cv
                ewb[j, sl] = jnp.where(ok, ewb[j, sl], 0.0)
                return 0

            lax.fori_loop(0, CHUNK // 16, xform, 0)
        dg = [None] * NBUF
        ds_ = [None] * NBUF
        for j0 in range(min(LOOKAHEAD + 1, ch)):
            dg[j0] = pltpu.async_copy(ytab.at[rowb.at[j0]], gath[j0],
                                      sg[j0])
        for j in range(ch):
            p = j % NBUF
            nxt = j + LOOKAHEAD + 1
            if nxt < ch:
                q = nxt % NBUF
                if ds_[q] is not None:
                    ds_[q].wait()
                    ds_[q] = None
                dg[q] = pltpu.async_copy(ytab.at[rowb.at[nxt]], gath[q],
                                         sg[q])
            dg[p].wait()

            def scale(k, _):
                ewv = ewb[j, pl.ds(k * 16, 16)]
                kb = k * 16
                for i in range(16):
                    gath[p][kb + i] = gath[p][kb + i] * ewv[i]
                return 0

            lax.fori_loop(0, CHUNK // 16, scale, 0)
            ds_[p] = pltpu.async_copy(gath[p], acc.at[colb.at[j]], ss[p],
                                      add=True)
        for d in ds_:
            if d is not None:
                d.wait()
        return 0

    lax.fori_loop(0, nblk, blk, 0)
    plsc.subcore_barrier()
    pltpu.sync_copy(acc.at[pl.ds(s * rt, rt)], zb)
    pltpu.sync_copy(zb, out.at[pl.ds(c * half + s * rt, rt)])


def _make_conv(n2, epc):
    half = n2 // NC
    rt = half // NS
    c_per_tile = epc // NS
    ch = _pick_div8(c_per_tile, 32)
    nblk = c_per_tile // ch
    mesh = plsc.VectorSubcoreMesh(core_axis_name="c", subcore_axis_name="s")
    return pl.kernel(
        functools.partial(_conv_body, half, rt, nblk, ch),
        out_type=jax.ShapeDtypeStruct((n2, 16), jnp.float32),
        mesh=mesh,
        compiler_params=pltpu.CompilerParams(use_tc_tiling_on_sc=False),
        scratch_types=[
            pltpu.VMEM_SHARED((half, 16), jnp.float32),
            pltpu.VMEM((ch, CHUNK), jnp.int32),
            pltpu.VMEM((ch, CHUNK), jnp.int32),
            pltpu.VMEM((ch, CHUNK), jnp.float32),
            pltpu.VMEM((CHUNK, 16), jnp.float32),
            pltpu.VMEM((CHUNK, 16), jnp.float32),
            pltpu.VMEM((CHUNK, 16), jnp.float32),
            pltpu.VMEM((CHUNK, 16), jnp.float32),
            pltpu.VMEM((rt, 16), jnp.float32),
        ] + [pltpu.SemaphoreType.DMA] * 9,
    )


def _tc1_body(dega, degb, x, dinv_o, y1_o):
    deg = dega[...] + degb[...] + 1.0
    dinv = lax.rsqrt(deg)
    dinv_o[...] = dinv
    y1_o[...] = x[...] * dinv


def _ln(h, g, b):
    m = jnp.mean(h, axis=-1, keepdims=True)
    v = jnp.mean((h - m) ** 2, axis=-1, keepdims=True)
    return (h - m) * lax.rsqrt(v + 1e-5) * g + b


def _tc2_body(a1, x, dinv, W1, b1, Wr1, br1, g1, be1, W2, Wr2, br2, b2,
              y2a_o, y2b_o, res2_o):
    dv = dinv[...]
    conv_in = dv * a1[...] + (dv * dv) * x[...]
    h = (jnp.dot(conv_in, W1[...], preferred_element_type=jnp.float32)
         + b1[...]
         + jnp.dot(x[...], Wr1[...], preferred_element_type=jnp.float32)
         + br1[...])
    x1 = jnp.maximum(_ln(h, g1[...], be1[...]), 0.0)
    y2 = dv * jnp.dot(x1, W2[...], preferred_element_type=jnp.float32)
    y2a_o[...] = y2[:, :16]
    y2b_o[...] = y2[:, 16:]
    res2_o[...] = (jnp.dot(x1, Wr2[...], preferred_element_type=jnp.float32)
                   + br2[...] + b2[...])


def _tc3_body(lo, hi, y2a, y2b, res2, dinv, g2, be2, Wfc, bfc, Wp, bp,
              log_o, emb_o):
    dv = dinv[...]
    a = jnp.concatenate([lo[...], hi[...]], axis=1)
    yy = jnp.concatenate([y2a[...], y2b[...]], axis=1)
    h = dv * a + dv * yy + res2[...]
    x2 = jnp.maximum(_ln(h, g2[...], be2[...]), 0.0)
    log_o[...] = (jnp.dot(x2, Wfc[...], preferred_element_type=jnp.float32)
                  + bfc[...])
    p = jnp.dot(x2, Wp[...], preferred_element_type=jnp.float32) + bp[...]
    nrm = jnp.sqrt(jnp.sum(p * p, axis=-1, keepdims=True))
    emb_o[...] = p / jnp.maximum(nrm, 1e-12)


def _row_spec(bn, f):
    return pl.BlockSpec((bn, f), lambda i: (i, 0))


def _full_spec(shape):
    nd = len(shape)
    return pl.BlockSpec(shape, lambda i: (0,) * nd)


def kernel(x, edge_index, edge_weight, W1, b1, Wr1, br1, g1, be1,
           W2, b2, Wr2, br2, g2, be2, Wfc, bfc, Wp, bp):
    n, in_dim = x.shape
    e = edge_weight.shape[0]
    hid = W1.shape[1]
    r = W2.shape[1]
    emb_dim = Wp.shape[1]

    grp = CHUNK * NC * NS * 8
    ep = ((e + grp - 1) // grp) * grp
    padn = ep - e
    row = edge_index[0]
    col = edge_index[1]
    if padn:
        pad_idx = (jnp.arange(padn, dtype=jnp.int32) * 97) % n
        row = jnp.concatenate([row, pad_idx])
        col = jnp.concatenate([col, pad_idx])
        ew = jnp.concatenate([edge_weight, jnp.zeros((padn,), jnp.float32)])
    else:
        ew = edge_weight
    epc = ep // CHUNK
    rowm = row.reshape(epc, CHUNK)
    colm = col.reshape(epc, CHUNK)
    ewm = ew.reshape(epc, CHUNK)

    npad = ((n + NS * 128 - 1) // (NS * 128)) * NS * 128
    n2 = ((n + 2 * NS * 8 - 1) // (2 * NS * 8)) * 2 * NS * 8

    deg2 = _make_deg(npad, epc)(colm, ewm)
    dega = deg2[0, :n].reshape(n, 1)
    degb = deg2[1, :n].reshape(n, 1)

    bn = _pick_div8(n, 2048)
    grid = n // bn
    dinv, y1 = pl.pallas_call(
        _tc1_body,
        grid=(grid,),
        in_specs=[_row_spec(bn, 1), _row_spec(bn, 1), _row_spec(bn, in_dim)],
        out_specs=[_row_spec(bn, 1), _row_spec(bn, in_dim)],
        out_shape=[jax.ShapeDtypeStruct((n, 1), jnp.float32),
                   jax.ShapeDtypeStruct((n, in_dim), jnp.float32)],
    )(dega, degb, x)

    conv = _make_conv(n2, epc)
    agg1 = conv(rowm, colm, ewm, y1)

    y2a, y2b, res2 = pl.pallas_call(
        _tc2_body,
        grid=(grid,),
        in_specs=[_row_spec(bn, in_dim),
                  _row_spec(bn, in_dim), _row_spec(bn, 1),
                  _full_spec((in_dim, hid)), _full_spec((1, hid)),
                  _full_spec((in_dim, hid)), _full_spec((1, hid)),
                  _full_spec((1, hid)), _full_spec((1, hid)),
                  _full_spec((hid, r)), _full_spec((hid, r)),
                  _full_spec((1, r)), _full_spec((1, r))],
        out_specs=[_row_spec(bn, 16), _row_spec(bn, 16),
                   _row_spec(bn, r)],
        out_shape=[jax.ShapeDtypeStruct((n, 16), jnp.float32),
                   jax.ShapeDtypeStruct((n, 16), jnp.float32),
                   jax.ShapeDtypeStruct((n, r), jnp.float32)],
    )(agg1, x, dinv,
      W1, b1.reshape(1, -1), Wr1, br1.reshape(1, -1),
      g1.reshape(1, -1), be1.reshape(1, -1),
      W2, Wr2, br2.reshape(1, -1), b2.reshape(1, -1))

    agg2lo = conv(rowm, colm, ewm, y2a)
    agg2hi = conv(rowm, colm, ewm, y2b)

    logits, emb = pl.pallas_call(
        _tc3_body,
        grid=(grid,),
        in_specs=[_row_spec(bn, 16), _row_spec(bn, 16),
                  _row_spec(bn, 16), _row_spec(bn, 16),
                  _row_spec(bn, r), _row_spec(bn, 1),
                  _full_spec((1, r)), _full_spec((1, r)),
                  _full_spec((r, 1)), _full_spec((1, 1)),
                  _full_spec((r, emb_dim)), _full_spec((1, emb_dim))],
        out_specs=[_row_spec(bn, 1), _row_spec(bn, emb_dim)],
        out_shape=[jax.ShapeDtypeStruct((n, 1), jnp.float32),
                   jax.ShapeDtypeStruct((n, emb_dim), jnp.float32)],
    )(agg2lo, agg2hi, y2a, y2b, res2, dinv,
      g2.reshape(1, -1), be2.reshape(1, -1),
      Wfc, bfc.reshape(1, -1), Wp, bp.reshape(1, -1))

    return logits.reshape(n), emb


# speedup vs baseline: 29.0119x; 1.1566x over previous
import functools

import jax
import jax.numpy as jnp
from jax import lax
from jax.experimental import pallas as pl
from jax.experimental.pallas import tpu as pltpu
from jax.experimental.pallas import tpu_sc as plsc

NC = 2
NS = 16
CHUNK = 128


def _pick_div8(n, cap):
    for d in range(cap - cap % 8, 0, -8):
        if n % d == 0:
            return d
    raise ValueError(f"no multiple-of-8 divisor of {n}")


def _zero_fill(ref, nrows):
    zv = jnp.zeros((16,), jnp.float32)

    def st(i, _):
        ref[i] = zv
        return 0

    lax.fori_loop(0, nrows, st, 0, unroll=8)


def _zero_fill_1d(ref, n):
    zv = jnp.zeros((16,), jnp.float32)

    def st(i, _):
        ref[pl.ds(i * 16, 16)] = zv
        return 0

    lax.fori_loop(0, n // 16, st, 0, unroll=8)


def _deg_body(snp, nblk, ch, colm, ewm, out, acc, colb, ewb, zb):
    c = lax.axis_index("c")
    s = lax.axis_index("s")
    w = c * NS + s
    _zero_fill_1d(zb, snp)
    pltpu.sync_copy(zb, acc.at[pl.ds(s * snp, snp)])
    plsc.subcore_barrier()

    def blk(b, _):
        bb = (w * nblk + b) * ch
        pltpu.sync_copy(colm.at[pl.ds(bb, ch)], colb)
        pltpu.sync_copy(ewm.at[pl.ds(bb, ch)], ewb)
        for j in range(ch):
            pltpu.sync_copy(ewb.at[j], acc.at[colb.at[j]], add=True)
        return 0

    lax.fori_loop(0, nblk, blk, 0)
    plsc.subcore_barrier()
    pltpu.sync_copy(acc.at[pl.ds(s * snp, snp)], zb)
    pltpu.sync_copy(zb, out.at[c, pl.ds(s * snp, snp)])


def _make_deg(npad, epc):
    snp = npad // NS
    c_per_tile = epc // (NC * NS)
    ch = _pick_div8(c_per_tile, 32)
    nblk = c_per_tile // ch
    mesh = plsc.VectorSubcoreMesh(core_axis_name="c", subcore_axis_name="s")
    return pl.kernel(
        functools.partial(_deg_body, snp, nblk, ch),
        out_type=jax.ShapeDtypeStruct((NC, npad), jnp.float32),
        mesh=mesh,
        compiler_params=pltpu.CompilerParams(use_tc_tiling_on_sc=False),
        scratch_types=[
            pltpu.VMEM_SHARED((npad,), jnp.float32),
            pltpu.VMEM((ch, CHUNK), jnp.int32),
            pltpu.VMEM((ch, CHUNK), jnp.float32),
            pltpu.VMEM((snp,), jnp.float32),
        ],
    )


NBUF = 6
LOOKAHEAD = 3


def _conv_body(half, rt, nblk, ch, rowm, colm, ewm, ytab, out,
               acc, rowb, colb, ewb, g0, g1, g2, g3, g4, g5, zb, sem_i,
               sg0, sg1, sg2, sg3, sg4, sg5, ss0, ss1, ss2, ss3, ss4, ss5):
    c = lax.axis_index("c")
    s = lax.axis_index("s")
    base = c * half
    _zero_fill(zb, rt)
    pltpu.sync_copy(zb, acc.at[pl.ds(s * rt, rt)])
    plsc.subcore_barrier()
    gath = (g0, g1, g2, g3, g4, g5)
    sg = (sg0, sg1, sg2, sg3, sg4, sg5)
    ss = (ss0, ss1, ss2, ss3, ss4, ss5)

    def blk(b, _):
        bb = (s * nblk + b) * ch
        d1 = pltpu.async_copy(rowm.at[pl.ds(bb, ch)], rowb, sem_i)
        d2 = pltpu.async_copy(colm.at[pl.ds(bb, ch)], colb, sem_i)
        d3 = pltpu.async_copy(ewm.at[pl.ds(bb, ch)], ewb, sem_i)
        d1.wait()
        dg = [None] * NBUF
        ds_ = [None] * NBUF
        for j0 in range(min(LOOKAHEAD + 1, ch)):
            dg[j0] = pltpu.async_copy(ytab.at[rowb.at[j0]], gath[j0],
                                      sg[j0])
        d2.wait()
        d3.wait()
        for j in range(ch):

            def xform(k, _):
                sl = pl.ds(k * 16, 16)
                cv = colb[j, sl] - base
                ok = (cv >= 0) & (cv < half)
                cv = jnp.where(cv < 0, cv + half, cv)
                cv = jnp.where(cv >= half, cv - half, cv)
                colb[j, sl] = cv
                ewb[j, sl] = jnp.where(ok, ewb[j, sl], 0.0)
                return 0

            lax.fori_loop(0, CHUNK // 16, xform, 0)
        for j in range(ch):
            p = j % NBUF
            nxt = j + LOOKAHEAD + 1
            if nxt < ch:
                q = nxt % NBUF
                if ds_[q] is not None:
                    ds_[q].wait()
                    ds_[q] = None
                dg[q] = pltpu.async_copy(ytab.at[rowb.at[nxt]], gath[q],
                                         sg[q])
            dg[p].wait()

            def scale(k, _):
                ewv = ewb[j, pl.ds(k * 16, 16)]
                kb = k * 16
                for i in range(16):
                    gath[p][kb + i] = gath[p][kb + i] * ewv[i]
                return 0

            lax.fori_loop(0, CHUNK // 16, scale, 0)
            ds_[p] = pltpu.async_copy(gath[p], acc.at[colb.at[j]], ss[p],
                                      add=True)
        for d in ds_:
            if d is not None:
                d.wait()
        return 0

    lax.fori_loop(0, nblk, blk, 0)
    plsc.subcore_barrier()
    pltpu.sync_copy(acc.at[pl.ds(s * rt, rt)], zb)
    pltpu.sync_copy(zb, out.at[pl.ds(c * half + s * rt, rt)])


def _make_conv(n2, epc):
    half = n2 // NC
    rt = half // NS
    c_per_tile = epc // NS
    ch = _pick_div8(c_per_tile, 32)
    nblk = c_per_tile // ch
    mesh = plsc.VectorSubcoreMesh(core_axis_name="c", subcore_axis_name="s")
    return pl.kernel(
        functools.partial(_conv_body, half, rt, nblk, ch),
        out_type=jax.ShapeDtypeStruct((n2, 16), jnp.float32),
        mesh=mesh,
        compiler_params=pltpu.CompilerParams(use_tc_tiling_on_sc=False),
        scratch_types=[
            pltpu.VMEM_SHARED((half, 16), jnp.float32),
            pltpu.VMEM((ch, CHUNK), jnp.int32),
            pltpu.VMEM((ch, CHUNK), jnp.int32),
            pltpu.VMEM((ch, CHUNK), jnp.float32),
        ] + [pltpu.VMEM((CHUNK, 16), jnp.float32)] * NBUF + [
            pltpu.VMEM((rt, 16), jnp.float32),
        ] + [pltpu.SemaphoreType.DMA] * (1 + 2 * NBUF),
    )


def _tc1_body(dega, degb, x, dinv_o, y1_o):
    deg = dega[...] + degb[...] + 1.0
    dinv = lax.rsqrt(deg)
    dinv_o[...] = dinv
    y1_o[...] = x[...] * dinv


def _ln(h, g, b):
    m = jnp.mean(h, axis=-1, keepdims=True)
    v = jnp.mean((h - m) ** 2, axis=-1, keepdims=True)
    return (h - m) * lax.rsqrt(v + 1e-5) * g + b


def _tc2_body(a1, x, dinv, W1, b1, Wr1, br1, g1, be1, W2, Wr2, br2, b2,
              y2a_o, y2b_o, res2_o):
    dv = dinv[...]
    conv_in = dv * a1[...] + (dv * dv) * x[...]
    h = (jnp.dot(conv_in, W1[...], preferred_element_type=jnp.float32)
         + b1[...]
         + jnp.dot(x[...], Wr1[...], preferred_element_type=jnp.float32)
         + br1[...])
    x1 = jnp.maximum(_ln(h, g1[...], be1[...]), 0.0)
    y2 = dv * jnp.dot(x1, W2[...], preferred_element_type=jnp.float32)
    y2a_o[...] = y2[:, :16]
    y2b_o[...] = y2[:, 16:]
    res2_o[...] = (jnp.dot(x1, Wr2[...], preferred_element_type=jnp.float32)
                   + br2[...] + b2[...])


def _tc3_body(lo, hi, y2a, y2b, res2, dinv, g2, be2, Wfc, bfc, Wp, bp,
              log_o, emb_o):
    dv = dinv[...]
    a = jnp.concatenate([lo[...], hi[...]], axis=1)
    yy = jnp.concatenate([y2a[...], y2b[...]], axis=1)
    h = dv * a + dv * yy + res2[...]
    x2 = jnp.maximum(_ln(h, g2[...], be2[...]), 0.0)
    log_o[...] = (jnp.dot(x2, Wfc[...], preferred_element_type=jnp.float32)
                  + bfc[...])
    p = jnp.dot(x2, Wp[...], preferred_element_type=jnp.float32) + bp[...]
    nrm = jnp.sqrt(jnp.sum(p * p, axis=-1, keepdims=True))
    emb_o[...] = p / jnp.maximum(nrm, 1e-12)


def _row_spec(bn, f):
    return pl.BlockSpec((bn, f), lambda i: (i, 0))


def _full_spec(shape):
    nd = len(shape)
    return pl.BlockSpec(shape, lambda i: (0,) * nd)


def kernel(x, edge_index, edge_weight, W1, b1, Wr1, br1, g1, be1,
           W2, b2, Wr2, br2, g2, be2, Wfc, bfc, Wp, bp):
    n, in_dim = x.shape
    e = edge_weight.shape[0]
    hid = W1.shape[1]
    r = W2.shape[1]
    emb_dim = Wp.shape[1]

    grp = CHUNK * NC * NS * 8
    ep = ((e + grp - 1) // grp) * grp
    padn = ep - e
    row = edge_index[0]
    col = edge_index[1]
    if padn:
        pad_idx = (jnp.arange(padn, dtype=jnp.int32) * 97) % n
        row = jnp.concatenate([row, pad_idx])
        col = jnp.concatenate([col, pad_idx])
        ew = jnp.concatenate([edge_weight, jnp.zeros((padn,), jnp.float32)])
    else:
        ew = edge_weight
    epc = ep // CHUNK
    rowm = row.reshape(epc, CHUNK)
    colm = col.reshape(epc, CHUNK)
    ewm = ew.reshape(epc, CHUNK)

    npad = ((n + NS * 128 - 1) // (NS * 128)) * NS * 128
    n2 = ((n + 2 * NS * 8 - 1) // (2 * NS * 8)) * 2 * NS * 8

    deg2 = _make_deg(npad, epc)(colm, ewm)
    dega = deg2[0, :n].reshape(n, 1)
    degb = deg2[1, :n].reshape(n, 1)

    bn = _pick_div8(n, 2048)
    grid = n // bn
    dinv, y1 = pl.pallas_call(
        _tc1_body,
        grid=(grid,),
        in_specs=[_row_spec(bn, 1), _row_spec(bn, 1), _row_spec(bn, in_dim)],
        out_specs=[_row_spec(bn, 1), _row_spec(bn, in_dim)],
        out_shape=[jax.ShapeDtypeStruct((n, 1), jnp.float32),
                   jax.ShapeDtypeStruct((n, in_dim), jnp.float32)],
    )(dega, degb, x)

    conv = _make_conv(n2, epc)
    agg1 = conv(rowm, colm, ewm, y1)

    y2a, y2b, res2 = pl.pallas_call(
        _tc2_body,
        grid=(grid,),
        in_specs=[_row_spec(bn, in_dim),
                  _row_spec(bn, in_dim), _row_spec(bn, 1),
                  _full_spec((in_dim, hid)), _full_spec((1, hid)),
                  _full_spec((in_dim, hid)), _full_spec((1, hid)),
                  _full_spec((1, hid)), _full_spec((1, hid)),
                  _full_spec((hid, r)), _full_spec((hid, r)),
                  _full_spec((1, r)), _full_spec((1, r))],
        out_specs=[_row_spec(bn, 16), _row_spec(bn, 16),
                   _row_spec(bn, r)],
        out_shape=[jax.ShapeDtypeStruct((n, 16), jnp.float32),
                   jax.ShapeDtypeStruct((n, 16), jnp.float32),
                   jax.ShapeDtypeStruct((n, r), jnp.float32)],
    )(agg1, x, dinv,
      W1, b1.reshape(1, -1), Wr1, br1.reshape(1, -1),
      g1.reshape(1, -1), be1.reshape(1, -1),
      W2, Wr2, br2.reshape(1, -1), b2.reshape(1, -1))

    agg2lo = conv(rowm, colm, ewm, y2a)
    agg2hi = conv(rowm, colm, ewm, y2b)

    logits, emb = pl.pallas_call(
        _tc3_body,
        grid=(grid,),
        in_specs=[_row_spec(bn, 16), _row_spec(bn, 16),
                  _row_spec(bn, 16), _row_spec(bn, 16),
                  _row_spec(bn, r), _row_spec(bn, 1),
                  _full_spec((1, r)), _full_spec((1, r)),
                  _full_spec((r, 1)), _full_spec((1, 1)),
                  _full_spec((r, emb_dim)), _full_spec((1, emb_dim))],
        out_specs=[_row_spec(bn, 1), _row_spec(bn, emb_dim)],
        out_shape=[jax.ShapeDtypeStruct((n, 1), jnp.float32),
                   jax.ShapeDtypeStruct((n, emb_dim), jnp.float32)],
    )(agg2lo, agg2hi, y2a, y2b, res2, dinv,
      g2.reshape(1, -1), be2.reshape(1, -1),
      Wfc, bfc.reshape(1, -1), Wp, bp.reshape(1, -1))

    return logits.reshape(n), emb


# speedup vs baseline: 29.9464x; 1.0322x over previous
import functools

import jax
import jax.numpy as jnp
from jax import lax
from jax.experimental import pallas as pl
from jax.experimental.pallas import tpu as pltpu
from jax.experimental.pallas import tpu_sc as plsc

NC = 2
NS = 16
CHUNK = 128


def _pick_div8(n, cap):
    for d in range(cap - cap % 8, 0, -8):
        if n % d == 0:
            return d
    raise ValueError(f"no multiple-of-8 divisor of {n}")


def _zero_fill(ref, nrows):
    zv = jnp.zeros((16,), jnp.float32)

    def st(i, _):
        ref[i] = zv
        return 0

    lax.fori_loop(0, nrows, st, 0, unroll=8)


def _zero_fill_1d(ref, n):
    zv = jnp.zeros((16,), jnp.float32)

    def st(i, _):
        ref[pl.ds(i * 16, 16)] = zv
        return 0

    lax.fori_loop(0, n // 16, st, 0, unroll=8)


def _deg_body(snp, nblk, ch, colm, ewm, out, acc, colb, ewb, zb, semd,
              *sems):
    c = lax.axis_index("c")
    s = lax.axis_index("s")
    w = c * NS + s
    _zero_fill_1d(zb, snp)
    pltpu.sync_copy(zb, acc.at[pl.ds(s * snp, snp)])
    plsc.subcore_barrier()

    def blk(b, _):
        bb = (w * nblk + b) * ch
        d1 = pltpu.async_copy(colm.at[pl.ds(bb, ch)], colb, semd)
        d2 = pltpu.async_copy(ewm.at[pl.ds(bb, ch)], ewb, semd)
        d1.wait()
        d2.wait()
        descs = [pltpu.async_copy(ewb.at[j], acc.at[colb.at[j]], sems[j],
                                  add=True)
                 for j in range(ch)]
        for d in descs:
            d.wait()
        return 0

    lax.fori_loop(0, nblk, blk, 0)
    plsc.subcore_barrier()
    pltpu.sync_copy(acc.at[pl.ds(s * snp, snp)], zb)
    pltpu.sync_copy(zb, out.at[c, pl.ds(s * snp, snp)])


def _make_deg(npad, epc):
    snp = npad // NS
    c_per_tile = epc // (NC * NS)
    ch = _pick_div8(c_per_tile, 32)
    nblk = c_per_tile // ch
    mesh = plsc.VectorSubcoreMesh(core_axis_name="c", subcore_axis_name="s")
    return pl.kernel(
        functools.partial(_deg_body, snp, nblk, ch),
        out_type=jax.ShapeDtypeStruct((NC, npad), jnp.float32),
        mesh=mesh,
        compiler_params=pltpu.CompilerParams(use_tc_tiling_on_sc=False),
        scratch_types=[
            pltpu.VMEM_SHARED((npad,), jnp.float32),
            pltpu.VMEM((ch, CHUNK), jnp.int32),
            pltpu.VMEM((ch, CHUNK), jnp.float32),
            pltpu.VMEM((snp,), jnp.float32),
        ] + [pltpu.SemaphoreType.DMA] * (1 + ch),
    )


NBUF = 6
LOOKAHEAD = 3


def _conv_body(half, rt, nblk, ch, rowm, colm, ewm, ytab, out,
               acc, rowb, colb, ewb, g0, g1, g2, g3, g4, g5, zb, sem_i,
               sg0, sg1, sg2, sg3, sg4, sg5, ss0, ss1, ss2, ss3, ss4, ss5):
    c = lax.axis_index("c")
    s = lax.axis_index("s")
    base = c * half
    _zero_fill(zb, rt)
    pltpu.sync_copy(zb, acc.at[pl.ds(s * rt, rt)])
    plsc.subcore_barrier()
    gath = (g0, g1, g2, g3, g4, g5)
    sg = (sg0, sg1, sg2, sg3, sg4, sg5)
    ss = (ss0, ss1, ss2, ss3, ss4, ss5)

    def blk(b, _):
        bb = (s * nblk + b) * ch
        d1 = pltpu.async_copy(rowm.at[pl.ds(bb, ch)], rowb, sem_i)
        d2 = pltpu.async_copy(colm.at[pl.ds(bb, ch)], colb, sem_i)
        d3 = pltpu.async_copy(ewm.at[pl.ds(bb, ch)], ewb, sem_i)
        d1.wait()
        dg = [None] * NBUF
        ds_ = [None] * NBUF
        for j0 in range(min(LOOKAHEAD + 1, ch)):
            dg[j0] = pltpu.async_copy(ytab.at[rowb.at[j0]], gath[j0],
                                      sg[j0])
        d2.wait()
        d3.wait()
        for j in range(ch):

            def xform(k, _):
                sl = pl.ds(k * 16, 16)
                cv = colb[j, sl] - base
                ok = (cv >= 0) & (cv < half)
                cv = jnp.where(cv < 0, cv + half, cv)
                cv = jnp.where(cv >= half, cv - half, cv)
                colb[j, sl] = cv
                ewb[j, sl] = jnp.where(ok, ewb[j, sl], 0.0)
                return 0

            lax.fori_loop(0, CHUNK // 16, xform, 0)
        for j in range(ch):
            p = j % NBUF
            nxt = j + LOOKAHEAD + 1
            if nxt < ch:
                q = nxt % NBUF
                if ds_[q] is not None:
                    ds_[q].wait()
                    ds_[q] = None
                dg[q] = pltpu.async_copy(ytab.at[rowb.at[nxt]], gath[q],
                                         sg[q])
            dg[p].wait()

            def scale(k, _):
                ewv = ewb[j, pl.ds(k * 16, 16)]
                kb = k * 16
                for i in range(16):
                    gath[p][kb + i] = gath[p][kb + i] * ewv[i]
                return 0

            lax.fori_loop(0, CHUNK // 16, scale, 0)
            ds_[p] = pltpu.async_copy(gath[p], acc.at[colb.at[j]], ss[p],
                                      add=True)
        for d in ds_:
            if d is not None:
                d.wait()
        return 0

    lax.fori_loop(0, nblk, blk, 0)
    plsc.subcore_barrier()
    pltpu.sync_copy(acc.at[pl.ds(s * rt, rt)], zb)
    pltpu.sync_copy(zb, out.at[pl.ds(c * half + s * rt, rt)])


def _make_conv(n2, epc):
    half = n2 // NC
    rt = half // NS
    c_per_tile = epc // NS
    ch = _pick_div8(c_per_tile, 32)
    nblk = c_per_tile // ch
    mesh = plsc.VectorSubcoreMesh(core_axis_name="c", subcore_axis_name="s")
    return pl.kernel(
        functools.partial(_conv_body, half, rt, nblk, ch),
        out_type=jax.ShapeDtypeStruct((n2, 16), jnp.float32),
        mesh=mesh,
        compiler_params=pltpu.CompilerParams(use_tc_tiling_on_sc=False),
        scratch_types=[
            pltpu.VMEM_SHARED((half, 16), jnp.float32),
            pltpu.VMEM((ch, CHUNK), jnp.int32),
            pltpu.VMEM((ch, CHUNK), jnp.int32),
            pltpu.VMEM((ch, CHUNK), jnp.float32),
        ] + [pltpu.VMEM((CHUNK, 16), jnp.float32)] * NBUF + [
            pltpu.VMEM((rt, 16), jnp.float32),
        ] + [pltpu.SemaphoreType.DMA] * (1 + 2 * NBUF),
    )


def _tc1_body(dega, degb, x, dinv_o, y1_o):
    deg = dega[...] + degb[...] + 1.0
    dinv = lax.rsqrt(deg)
    dinv_o[...] = dinv
    y1_o[...] = x[...] * dinv


def _ln(h, g, b):
    m = jnp.mean(h, axis=-1, keepdims=True)
    v = jnp.mean((h - m) ** 2, axis=-1, keepdims=True)
    return (h - m) * lax.rsqrt(v + 1e-5) * g + b


def _tc2_body(a1, x, dinv, W1, b1, Wr1, br1, g1, be1, W2, Wr2, br2, b2,
              y2a_o, y2b_o, res2_o):
    dv = dinv[...]
    conv_in = dv * a1[...] + (dv * dv) * x[...]
    h = (jnp.dot(conv_in, W1[...], preferred_element_type=jnp.float32)
         + b1[...]
         + jnp.dot(x[...], Wr1[...], preferred_element_type=jnp.float32)
         + br1[...])
    x1 = jnp.maximum(_ln(h, g1[...], be1[...]), 0.0)
    y2 = dv * jnp.dot(x1, W2[...], preferred_element_type=jnp.float32)
    y2a_o[...] = y2[:, :16]
    y2b_o[...] = y2[:, 16:]
    res2_o[...] = (jnp.dot(x1, Wr2[...], preferred_element_type=jnp.float32)
                   + br2[...] + b2[...])


def _tc3_body(lo, hi, y2a, y2b, res2, dinv, g2, be2, Wfc, bfc, Wp, bp,
              log_o, emb_o):
    dv = dinv[...]
    a = jnp.concatenate([lo[...], hi[...]], axis=1)
    yy = jnp.concatenate([y2a[...], y2b[...]], axis=1)
    h = dv * a + dv * yy + res2[...]
    x2 = jnp.maximum(_ln(h, g2[...], be2[...]), 0.0)
    log_o[...] = (jnp.dot(x2, Wfc[...], preferred_element_type=jnp.float32)
                  + bfc[...])
    p = jnp.dot(x2, Wp[...], preferred_element_type=jnp.float32) + bp[...]
    nrm = jnp.sqrt(jnp.sum(p * p, axis=-1, keepdims=True))
    emb_o[...] = p / jnp.maximum(nrm, 1e-12)


def _row_spec(bn, f):
    return pl.BlockSpec((bn, f), lambda i: (i, 0))


def _full_spec(shape):
    nd = len(shape)
    return pl.BlockSpec(shape, lambda i: (0,) * nd)


def kernel(x, edge_index, edge_weight, W1, b1, Wr1, br1, g1, be1,
           W2, b2, Wr2, br2, g2, be2, Wfc, bfc, Wp, bp):
    n, in_dim = x.shape
    e = edge_weight.shape[0]
    hid = W1.shape[1]
    r = W2.shape[1]
    emb_dim = Wp.shape[1]

    grp = CHUNK * NC * NS * 8
    ep = ((e + grp - 1) // grp) * grp
    padn = ep - e
    row = edge_index[0]
    col = edge_index[1]
    if padn:
        pad_idx = (jnp.arange(padn, dtype=jnp.int32) * 97) % n
        row = jnp.concatenate([row, pad_idx])
        col = jnp.concatenate([col, pad_idx])
        ew = jnp.concatenate([edge_weight, jnp.zeros((padn,), jnp.float32)])
    else:
        ew = edge_weight
    epc = ep // CHUNK
    rowm = row.reshape(epc, CHUNK)
    colm = col.reshape(epc, CHUNK)
    ewm = ew.reshape(epc, CHUNK)

    npad = ((n + NS * 128 - 1) // (NS * 128)) * NS * 128
    n2 = ((n + 2 * NS * 8 - 1) // (2 * NS * 8)) * 2 * NS * 8

    deg2 = _make_deg(npad, epc)(colm, ewm)
    dega = deg2[0, :n].reshape(n, 1)
    degb = deg2[1, :n].reshape(n, 1)

    bn = _pick_div8(n, 2048)
    grid = n // bn
    dinv, y1 = pl.pallas_call(
        _tc1_body,
        grid=(grid,),
        in_specs=[_row_spec(bn, 1), _row_spec(bn, 1), _row_spec(bn, in_dim)],
        out_specs=[_row_spec(bn, 1), _row_spec(bn, in_dim)],
        out_shape=[jax.ShapeDtypeStruct((n, 1), jnp.float32),
                   jax.ShapeDtypeStruct((n, in_dim), jnp.float32)],
    )(dega, degb, x)

    conv = _make_conv(n2, epc)
    agg1 = conv(rowm, colm, ewm, y1)

    y2a, y2b, res2 = pl.pallas_call(
        _tc2_body,
        grid=(grid,),
        in_specs=[_row_spec(bn, in_dim),
                  _row_spec(bn, in_dim), _row_spec(bn, 1),
                  _full_spec((in_dim, hid)), _full_spec((1, hid)),
                  _full_spec((in_dim, hid)), _full_spec((1, hid)),
                  _full_spec((1, hid)), _full_spec((1, hid)),
                  _full_spec((hid, r)), _full_spec((hid, r)),
                  _full_spec((1, r)), _full_spec((1, r))],
        out_specs=[_row_spec(bn, 16), _row_spec(bn, 16),
                   _row_spec(bn, r)],
        out_shape=[jax.ShapeDtypeStruct((n, 16), jnp.float32),
                   jax.ShapeDtypeStruct((n, 16), jnp.float32),
                   jax.ShapeDtypeStruct((n, r), jnp.float32)],
    )(agg1, x, dinv,
      W1, b1.reshape(1, -1), Wr1, br1.reshape(1, -1),
      g1.reshape(1, -1), be1.reshape(1, -1),
      W2, Wr2, br2.reshape(1, -1), b2.reshape(1, -1))

    agg2lo = conv(rowm, colm, ewm, y2a)
    agg2hi = conv(rowm, colm, ewm, y2b)

    logits, emb = pl.pallas_call(
        _tc3_body,
        grid=(grid,),
        in_specs=[_row_spec(bn, 16), _row_spec(bn, 16),
                  _row_spec(bn, 16), _row_spec(bn, 16),
                  _row_spec(bn, r), _row_spec(bn, 1),
                  _full_spec((1, r)), _full_spec((1, r)),
                  _full_spec((r, 1)), _full_spec((1, 1)),
                  _full_spec((r, emb_dim)), _full_spec((1, emb_dim))],
        out_specs=[_row_spec(bn, 1), _row_spec(bn, emb_dim)],
        out_shape=[jax.ShapeDtypeStruct((n, 1), jnp.float32),
                   jax.ShapeDtypeStruct((n, emb_dim), jnp.float32)],
    )(agg2lo, agg2hi, y2a, y2b, res2, dinv,
      g2.reshape(1, -1), be2.reshape(1, -1),
      Wfc, bfc.reshape(1, -1), Wp, bp.reshape(1, -1))

    return logits.reshape(n), emb
